# baseline (speedup 1.0000x reference)
import functools

import numpy as np
import jax
import jax.numpy as jnp
from jax import lax
from jax.experimental import pallas as pl
from jax.experimental.pallas import tpu as pltpu
from jax.experimental.pallas import tpu_sc as plsc

_D = 256
_ROWS = 50
_L = 16


@functools.lru_cache(maxsize=None)
def _make_sc_kernel(B: int, h: int, w: int):
    assert h == w and h % _L == 0
    HW = h * w
    NB = _L * HW
    out_words = B * 2 * _D * HW
    mesh = plsc.VectorSubcoreMesh(core_axis_name="c", subcore_axis_name="s")

    @functools.partial(
        pl.kernel,
        mesh=mesh,
        out_type=jax.ShapeDtypeStruct((out_words,), jnp.float32),
        compiler_params=pltpu.CompilerParams(needs_layout_passes=False),
        scratch_types=[
            pltpu.VMEM((_ROWS * _D,), jnp.float32),
            pltpu.VMEM((_L * h + 8,), jnp.float32),
            pltpu.VMEM((NB,), jnp.float32),
            pltpu.SemaphoreType.DMA,
        ],
    )
    def body(colf, rowf, out, tbl_v, e_v, buf_v, sem):
        wid = lax.axis_index("s") * 2 + lax.axis_index("c")
        half = wid // 16
        grp = wid % 16
        cbase = grp * _L

        @pl.when(half == 0)
        def _():
            pltpu.sync_copy(colf, tbl_v)

        @pl.when(half == 1)
        def _():
            pltpu.sync_copy(rowf, tbl_v)

        iota = lax.iota(jnp.int32, _L)
        izero = lax.shift_right_logical(iota, 4)
        scale = np.float32(49.0) / np.float32(h)
        for ch in range(h // _L):
            coordv = (iota + ch * _L).astype(jnp.float32) * scale
            fi_i = coordv.astype(jnp.int32)
            deltav = coordv - fi_i.astype(jnp.float32)
            wfv = np.float32(1.0) - deltav
            ci_i = jnp.minimum(fi_i + 1, _ROWS - 1)
            fiv = fi_i * _D
            civ = ci_i * _D
            for cl in range(_L):
                c = cbase + cl
                vf = plsc.load_gather(tbl_v, [fiv + c])
                vc = plsc.load_gather(tbl_v, [civ + c])
                e_v[pl.ds(cl * h + ch * _L + 8, _L)] = wfv * vf + deltav * vc

        def expand_x(cl):
            for ch in range(w // _L):
                v = e_v[pl.ds(cl * h + ch * _L + 8, _L)]
                for rep in range(h):
                    buf_v[pl.ds(cl * HW + rep * w + ch * _L, _L)] = v

        def expand_y(cl):
            for hh in range(h):
                v = plsc.load_gather(e_v, [izero + (cl * h + hh + 8)])
                for ch in range(w // _L):
                    buf_v[pl.ds(cl * HW + hh * w + ch * _L, _L)] = v

        def fire(part):
            rows = half * _D + cbase + part * (_L // 2)
            n = (_L // 2) * HW
            copies = []
            for b in range(B):
                dst = (b * 2 * _D + rows) * HW
                copies.append(pltpu.async_copy(
                    buf_v.at[pl.ds(part * n, n)], out.at[pl.ds(dst, n)], sem))
            return copies

        pending = []
        for part in range(2):
            cls = range(part * (_L // 2), (part + 1) * (_L // 2))

            @pl.when(half == 0)
            def _(cls=cls):
                for cl in cls:
                    expand_x(cl)

            @pl.when(half == 1)
            def _(cls=cls):
                for cl in cls:
                    expand_y(cl)

            pending += fire(part)
        for cp in pending:
            cp.wait()

    return body


def kernel(x, row_embed, col_embed):
    B = x.shape[0]
    h, w = x.shape[-2], x.shape[-1]
    out = _make_sc_kernel(B, h, w)(
        col_embed.reshape(-1), row_embed.reshape(-1))
    return out.reshape(B, 2 * _D, h, w)


# speedup vs baseline: 2.4309x; 2.4309x over previous
import functools

import numpy as np
import jax
import jax.numpy as jnp
from jax import lax
from jax.experimental import pallas as pl
from jax.experimental.pallas import tpu as pltpu
from jax.experimental.pallas import tpu_sc as plsc

_D = 256
_ROWS = 50
_L = 16
_TS, _TLN = 8, 128


@functools.lru_cache(maxsize=None)
def _make_sc_kernel(B: int, h: int, w: int):
    assert h == w == 32, "kernel is specialized to the problem's 32x32 grid"
    WT = w // _TS
    CT = 2 * _D // _TLN
    BLK = WT * CT * _TS * _TLN
    out_words = B * h * BLK
    mesh = plsc.VectorSubcoreMesh(core_axis_name="c", subcore_axis_name="s")

    coord = np.arange(w, dtype=np.float32) / np.float32(w) * np.float32(49.0)
    fi_np = np.floor(coord).astype(np.int32)
    ci_np = np.minimum(fi_np + 1, _ROWS - 1)
    wc_np = coord - np.floor(coord)
    wf_np = np.float32(1.0) - wc_np

    @functools.partial(
        pl.kernel,
        mesh=mesh,
        out_type=jax.ShapeDtypeStruct((out_words,), jnp.float32),
        compiler_params=pltpu.CompilerParams(needs_layout_passes=False),
        scratch_types=[
            pltpu.VMEM((_ROWS * _D,), jnp.float32),
            pltpu.VMEM((2 * _D,), jnp.float32),
            pltpu.VMEM((BLK,), jnp.float32),
            pltpu.SemaphoreType.DMA,
        ],
    )
    def body(colf, rowf, out, tbl_v, yrow_v, blk_v, sem):
        hh = lax.axis_index("s") * 2 + lax.axis_index("c")

        pltpu.sync_copy(colf, tbl_v)
        coords = hh.astype(jnp.float32) * (np.float32(49.0) / np.float32(h))
        fis = coords.astype(jnp.int32)
        wcs = coords - fis.astype(jnp.float32)
        wfs = np.float32(1.0) - wcs
        pltpu.sync_copy(rowf.at[pl.ds(fis * _D, 2 * _D)], yrow_v)

        yv = [yrow_v[pl.ds(k * _L, _L)] * wfs
              + yrow_v[pl.ds(_D + k * _L, _L)] * wcs
              for k in range(_D // _L)]

        pending = []
        for part in range(2):
            for wt in range(part * WT // 2, (part + 1) * WT // 2):
                for ws in range(_TS):
                    ww = wt * _TS + ws
                    fo, co = int(fi_np[ww]) * _D, int(ci_np[ww]) * _D
                    wf, wc = float(wf_np[ww]), float(wc_np[ww])
                    for ct in range(CT):
                        pos = ((wt * CT + ct) * _TS + ws) * _TLN
                        for k in range(_TLN // _L):
                            c0 = (ct % 2) * _TLN + k * _L
                            dst = pl.ds(pos + k * _L, _L)
                            if ct < 2:
                                vf = tbl_v[pl.ds(fo + c0, _L)]
                                vc = tbl_v[pl.ds(co + c0, _L)]
                                blk_v[dst] = vf * wf + vc * wc
                            else:
                                blk_v[dst] = yv[c0 // _L]
            half = BLK // 2
            for b in range(B):
                base = (b * h + hh) * BLK + part * half
                pending.append(pltpu.async_copy(
                    blk_v.at[pl.ds(part * half, half)],
                    out.at[pl.ds(base, half)], sem))
        for cp in pending:
            cp.wait()

    return body


def kernel(x, row_embed, col_embed):
    B = x.shape[0]
    h, w = x.shape[-2], x.shape[-1]
    flat = _make_sc_kernel(B, h, w)(
        col_embed.reshape(-1), row_embed.reshape(-1))
    r6 = flat.reshape(B, h, w // _TS, 2 * _D // _TLN, _TS, _TLN)
    return r6.transpose(0, 3, 5, 1, 2, 4).reshape(B, 2 * _D, h, w)
